# baseline (speedup 1.0000x reference)
import math

import jax
import jax.numpy as jnp
from jax.experimental import pallas as pl
from jax.experimental.pallas import tpu as pltpu

NODE_DIM = 256
FT_DIM = 256
HIDDEN_DIM = 256
TIME_EMB_DIM = 128
T_STEPS = 8
TILE = 2048
MAX_POS = 10000.0


def _fold_weights_kernel(atom_ref, w1_ref, w2_ref, b1_ref, b2_ref,
                         a2_ref, ct_ref, bc_ref):
    w1a = w1_ref[:, :NODE_DIM]
    w1f = w1_ref[:, NODE_DIM:]
    w2h = w2_ref[:, :HIDDEN_DIM]
    atom_p = jnp.concatenate(
        [atom_ref[...],
         jnp.zeros((128 - atom_ref.shape[0], NODE_DIM), jnp.float32)], axis=0)
    a1 = jax.lax.dot_general(atom_p, w1a, (((1,), (1,)), ((), ())),
                             preferred_element_type=jnp.float32)
    a2 = jax.lax.dot_general(a1, w2h, (((1,), (1,)), ((), ())),
                             preferred_element_type=jnp.float32)
    a2_ref[...] = a2.astype(jnp.bfloat16)
    ct_ref[...] = jnp.dot(w2h, w1f,
                          preferred_element_type=jnp.float32).astype(jnp.bfloat16)
    b1v = b1_ref[...].reshape(1, HIDDEN_DIM)
    b2v = b2_ref[...].reshape(1, HIDDEN_DIM)
    bc_ref[...] = jax.lax.dot_general(b1v, w2h, (((1,), (1,)), ((), ())),
                                      preferred_element_type=jnp.float32) + b2v


def _main_kernel(h_ref, dt_ref, f_ref, w2_ref, a2_ref, ct_ref, bc_ref,
                 out_ref):
    tile = f_ref.shape[0]
    hv = jnp.transpose(h_ref[...].reshape(1, tile), (1, 0))
    lane = jax.lax.broadcasted_iota(jnp.int32, (tile, 128), 1)
    onehot = (lane == hv).astype(jnp.bfloat16)
    o = jnp.dot(onehot, a2_ref[...], preferred_element_type=jnp.float32)
    o += jax.lax.dot_general(f_ref[...].astype(jnp.bfloat16), ct_ref[...],
                             (((1,), (1,)), ((), ())),
                             preferred_element_type=jnp.float32)
    half = TIME_EMB_DIM // 2
    coef = -math.log(MAX_POS) / (half - 1)
    j = jax.lax.broadcasted_iota(jnp.int32, (tile, half), 1).astype(jnp.float32)
    freq = jnp.exp(j * coef)
    dtv = jnp.transpose(dt_ref[...].reshape(1, tile), (1, 0)).astype(jnp.float32)
    args = dtv * freq
    temb = jnp.concatenate([jnp.sin(args), jnp.cos(args)], axis=1)
    w2t = w2_ref[:, HIDDEN_DIM:]
    o += jax.lax.dot_general(temb.astype(jnp.bfloat16),
                             w2t.astype(jnp.bfloat16),
                             (((1,), (1,)), ((), ())),
                             preferred_element_type=jnp.float32)
    o += bc_ref[...]
    out_ref[...] = jnp.broadcast_to(o[:, None, :], (tile, T_STEPS, HIDDEN_DIM))


@jax.jit
def kernel(diffusion_t, x, h, f, edge_index, edge_attr, batch, atom_emb,
           emb_lin_W, emb_lin_b, edge_emb_table, input_lin_W, input_lin_b,
           cond_emb_table):
    bn = f.shape[0]
    a2, ct, bc = pl.pallas_call(
        _fold_weights_kernel,
        out_shape=(
            jax.ShapeDtypeStruct((128, HIDDEN_DIM), jnp.bfloat16),
            jax.ShapeDtypeStruct((HIDDEN_DIM, FT_DIM), jnp.bfloat16),
            jax.ShapeDtypeStruct((1, HIDDEN_DIM), jnp.float32),
        ),
    )(atom_emb, emb_lin_W, input_lin_W, emb_lin_b, input_lin_b)

    grid = pl.cdiv(bn, TILE)
    out_flat = pl.pallas_call(
        _main_kernel,
        grid=(grid,),
        in_specs=[
            pl.BlockSpec((TILE,), lambda g: (g,)),
            pl.BlockSpec((TILE,), lambda g: (g,)),
            pl.BlockSpec((TILE, FT_DIM), lambda g: (g, 0)),
            pl.BlockSpec((HIDDEN_DIM, NODE_DIM + TIME_EMB_DIM),
                         lambda g: (0, 0)),
            pl.BlockSpec((128, HIDDEN_DIM), lambda g: (0, 0)),
            pl.BlockSpec((HIDDEN_DIM, FT_DIM), lambda g: (0, 0)),
            pl.BlockSpec((1, HIDDEN_DIM), lambda g: (0, 0)),
        ],
        out_specs=pl.BlockSpec((TILE, T_STEPS, HIDDEN_DIM), lambda g: (g, 0, 0)),
        out_shape=jax.ShapeDtypeStruct((bn, T_STEPS, HIDDEN_DIM), jnp.float32),
        compiler_params=pltpu.CompilerParams(
            dimension_semantics=("parallel",)),
    )(h, diffusion_t.astype(jnp.int32), f, input_lin_W, a2, ct, bc)

    return jnp.transpose(out_flat, (0, 2, 1))


# speedup vs baseline: 1.2523x; 1.2523x over previous
import math

import jax
import jax.numpy as jnp
from jax.experimental import pallas as pl
from jax.experimental.pallas import tpu as pltpu

NODE_DIM = 256
FT_DIM = 256
HIDDEN_DIM = 256
TIME_EMB_DIM = 128
T_STEPS = 8
TILE = 1024
MAX_POS = 10000.0


def _fold_weights_kernel(atom_ref, w1_ref, w2_ref, b1_ref, b2_ref,
                         a2_ref, ct_ref, tas_ref, tac_ref, tbs_ref, tbc_ref):
    w1a = w1_ref[:, :NODE_DIM]
    w1f = w1_ref[:, NODE_DIM:]
    w2h = w2_ref[:, :HIDDEN_DIM]
    atom_p = jnp.concatenate(
        [atom_ref[...],
         jnp.zeros((128 - atom_ref.shape[0], NODE_DIM), jnp.float32)], axis=0)
    a1 = jax.lax.dot_general(atom_p, w1a, (((1,), (1,)), ((), ())),
                             preferred_element_type=jnp.float32)
    a2 = jax.lax.dot_general(a1, w2h, (((1,), (1,)), ((), ())),
                             preferred_element_type=jnp.float32)
    b1v = b1_ref[...].reshape(1, HIDDEN_DIM)
    b2v = b2_ref[...].reshape(1, HIDDEN_DIM)
    bc = jax.lax.dot_general(b1v, w2h, (((1,), (1,)), ((), ())),
                             preferred_element_type=jnp.float32) + b2v
    a2_ref[...] = (a2 + bc).astype(jnp.bfloat16)
    ct_ref[...] = jnp.dot(w2h, w1f,
                          preferred_element_type=jnp.float32).astype(jnp.bfloat16)
    half = TIME_EMB_DIM // 2
    coef = -math.log(MAX_POS) / (half - 1)
    fr = jnp.exp(
        jax.lax.broadcasted_iota(jnp.int32, (128, half), 1).astype(jnp.float32)
        * coef)
    qs = jax.lax.broadcasted_iota(jnp.int32, (128, half), 0).astype(jnp.float32)
    arg_a = (qs * 32.0) * fr
    arg_b = qs * fr
    tas_ref[...] = jnp.sin(arg_a)
    tac_ref[...] = jnp.cos(arg_a)
    tbs_ref[...] = jnp.sin(arg_b)
    tbc_ref[...] = jnp.cos(arg_b)


def _main_kernel(h_ref, dt_ref, f_ref, w2_ref, a2_ref, ct_ref,
                 tas_ref, tac_ref, tbs_ref, tbc_ref, out_ref):
    tile = f_ref.shape[0]
    hv = jnp.transpose(h_ref[...].reshape(1, tile), (1, 0))
    lane = jax.lax.broadcasted_iota(jnp.int32, (tile, 128), 1)
    onehot = (lane == hv).astype(jnp.bfloat16)
    o = jnp.dot(onehot, a2_ref[...], preferred_element_type=jnp.float32)
    o += jax.lax.dot_general(f_ref[...].astype(jnp.bfloat16), ct_ref[...],
                             (((1,), (1,)), ((), ())),
                             preferred_element_type=jnp.float32)
    dts = jnp.transpose(dt_ref[...].reshape(1, tile), (1, 0))
    q = dts >> 5
    r = dts & 31
    ohq = (lane == q).astype(jnp.float32)
    ohr = (lane == r).astype(jnp.float32)
    sa = jnp.dot(ohq, tas_ref[...], preferred_element_type=jnp.float32)
    ca = jnp.dot(ohq, tac_ref[...], preferred_element_type=jnp.float32)
    sb = jnp.dot(ohr, tbs_ref[...], preferred_element_type=jnp.float32)
    cb = jnp.dot(ohr, tbc_ref[...], preferred_element_type=jnp.float32)
    temb = jnp.concatenate([sa * cb + ca * sb, ca * cb - sa * sb], axis=1)
    w2t = w2_ref[:, HIDDEN_DIM:]
    o += jax.lax.dot_general(temb.astype(jnp.bfloat16),
                             w2t.astype(jnp.bfloat16),
                             (((1,), (1,)), ((), ())),
                             preferred_element_type=jnp.float32)
    out_ref[...] = jnp.broadcast_to(o[:, None, :], (tile, T_STEPS, HIDDEN_DIM))


@jax.jit
def kernel(diffusion_t, x, h, f, edge_index, edge_attr, batch, atom_emb,
           emb_lin_W, emb_lin_b, edge_emb_table, input_lin_W, input_lin_b,
           cond_emb_table):
    bn = f.shape[0]
    a2, ct, tas, tac, tbs, tbc = pl.pallas_call(
        _fold_weights_kernel,
        out_shape=(
            jax.ShapeDtypeStruct((128, HIDDEN_DIM), jnp.bfloat16),
            jax.ShapeDtypeStruct((HIDDEN_DIM, FT_DIM), jnp.bfloat16),
            jax.ShapeDtypeStruct((128, TIME_EMB_DIM // 2), jnp.float32),
            jax.ShapeDtypeStruct((128, TIME_EMB_DIM // 2), jnp.float32),
            jax.ShapeDtypeStruct((128, TIME_EMB_DIM // 2), jnp.float32),
            jax.ShapeDtypeStruct((128, TIME_EMB_DIM // 2), jnp.float32),
        ),
    )(atom_emb, emb_lin_W, input_lin_W, emb_lin_b, input_lin_b)

    grid = pl.cdiv(bn, TILE)
    out_flat = pl.pallas_call(
        _main_kernel,
        grid=(grid,),
        in_specs=[
            pl.BlockSpec((TILE,), lambda g: (g,)),
            pl.BlockSpec((TILE,), lambda g: (g,)),
            pl.BlockSpec((TILE, FT_DIM), lambda g: (g, 0)),
            pl.BlockSpec((HIDDEN_DIM, NODE_DIM + TIME_EMB_DIM),
                         lambda g: (0, 0)),
            pl.BlockSpec((128, HIDDEN_DIM), lambda g: (0, 0)),
            pl.BlockSpec((HIDDEN_DIM, FT_DIM), lambda g: (0, 0)),
            pl.BlockSpec((128, TIME_EMB_DIM // 2), lambda g: (0, 0)),
            pl.BlockSpec((128, TIME_EMB_DIM // 2), lambda g: (0, 0)),
            pl.BlockSpec((128, TIME_EMB_DIM // 2), lambda g: (0, 0)),
            pl.BlockSpec((128, TIME_EMB_DIM // 2), lambda g: (0, 0)),
        ],
        out_specs=pl.BlockSpec((TILE, T_STEPS, HIDDEN_DIM), lambda g: (g, 0, 0)),
        out_shape=jax.ShapeDtypeStruct((bn, T_STEPS, HIDDEN_DIM), jnp.float32),
        compiler_params=pltpu.CompilerParams(
            dimension_semantics=("parallel",)),
    )(h, diffusion_t.astype(jnp.int32), f, input_lin_W, a2, ct,
      tas, tac, tbs, tbc)

    return jnp.transpose(out_flat, (0, 2, 1))
